# Initial kernel scaffold; baseline (speedup 1.0000x reference)
import jax
import jax.numpy as jnp
from jax.experimental import pallas as pl


def kernel(x, edge_index, length, dim, W, b):
    raise NotImplementedError("write your pallas kernel here")


# speedup vs baseline: 27.2064x; 27.2064x over previous
import functools

import jax
import jax.numpy as jnp
from jax import lax
from jax.experimental import pallas as pl
from jax.experimental.pallas import tpu as pltpu
from jax.experimental.pallas import tpu_sc as plsc

N = 10000
E = 320000
D = 128

NC = 2
NS = 16
NW = NC * NS
EPT = E // NW
K = 80
NB = EPT // K
CH = 1000
NCH = N // CH

_mesh = plsc.VectorSubcoreMesh(core_axis_name="c", subcore_axis_name="s")


NP = 10240
HR = NP // D


@functools.partial(
    pl.kernel,
    out_type=jax.ShapeDtypeStruct((NC, HR, D), jnp.float32),
    mesh=_mesh,
    compiler_params=pltpu.CompilerParams(needs_layout_passes=False),
    scratch_types=[
        pltpu.VMEM((EPT,), jnp.int32),
        pltpu.VMEM((HR, D), jnp.float32),
        pltpu.VMEM((1, HR), jnp.int32),
        pltpu.VMEM_SHARED((HR, D), jnp.float32),
    ],
)
def _deg_kernel(col_hbm, ridx_hbm, zrows_hbm, out_hbm, colb, hist, ridx, dacc):
    c = lax.axis_index("c")
    s = lax.axis_index("s")
    wid = s * NC + c

    @pl.when(s == 0)
    def _init():
        pltpu.sync_copy(zrows_hbm, dacc)

    pltpu.sync_copy(zrows_hbm, hist)
    pltpu.sync_copy(col_hbm.at[wid], colb)
    pltpu.sync_copy(ridx_hbm, ridx)
    ones16 = jnp.ones((16,), jnp.float32)

    def body(i, carry):
        idx = colb[pl.ds(i * 16, 16)]
        r = lax.shift_right_logical(idx, 7)
        q = lax.bitwise_and(idx, 127)
        plsc.addupdate_scatter(hist, [r, q], ones16)
        return carry

    lax.fori_loop(0, EPT // 16, body, 0)
    plsc.subcore_barrier()
    pltpu.sync_copy(hist, dacc.at[ridx.at[0]], add=True)
    plsc.subcore_barrier()

    @pl.when(s == 0)
    def _drain():
        pltpu.sync_copy(dacc, out_hbm.at[c])


@functools.partial(
    pl.kernel,
    out_type=jax.ShapeDtypeStruct((NC, N, D), jnp.float32),
    mesh=_mesh,
    scratch_types=[
        pltpu.VMEM((NB, K), jnp.int32),
        pltpu.VMEM((NB, K), jnp.int32),
        pltpu.VMEM((K, D), jnp.float32),
        pltpu.VMEM_SHARED((N, D), jnp.float32),
        pltpu.SemaphoreType.DMA,
    ],
)
def _scatter_kernel(row_hbm, col_hbm, hs_hbm, zeros_hbm, out_hbm,
                    rowv, colv, buf, acc, sem):
    c = lax.axis_index("c")
    s = lax.axis_index("s")
    wid = s * NC + c

    @pl.when(s < NCH)
    def _init():
        pltpu.sync_copy(zeros_hbm.at[pl.ds(s * CH, CH)],
                        acc.at[pl.ds(s * CH, CH)])

    pltpu.sync_copy(row_hbm.at[wid], rowv)
    pltpu.sync_copy(col_hbm.at[wid], colv)
    plsc.subcore_barrier()

    def body(j, carry):
        pltpu.async_copy(hs_hbm.at[rowv.at[j]], buf, sem).wait()
        pltpu.sync_copy(buf, acc.at[colv.at[j]], add=True)
        return carry

    lax.fori_loop(0, NB, body, 0)
    plsc.subcore_barrier()

    @pl.when(s < NCH)
    def _drain():
        pltpu.sync_copy(acc.at[pl.ds(s * CH, CH)],
                        out_hbm.at[c, pl.ds(s * CH, CH)])


def _mm_body(x_ref, w_ref, o_ref):
    o_ref[...] = jnp.dot(x_ref[...], w_ref[...],
                         preferred_element_type=jnp.float32)


def _scale_body(h_ref, dg_ref, o_ref):
    o_ref[...] = h_ref[...] * lax.rsqrt(dg_ref[...])


def _finish_body(p_ref, hs_ref, dg_ref, b_ref, o_ref):
    acc = p_ref[0] + p_ref[1] + hs_ref[...]
    t = acc * lax.rsqrt(dg_ref[...]) + b_ref[...]
    t = jnp.maximum(t, 0.0)
    m = jnp.max(t, axis=1, keepdims=True)
    e = jnp.exp(t - m)
    o_ref[...] = e / jnp.sum(e, axis=1, keepdims=True)


_RB = 1000
_G = N // _RB


def kernel(x, edge_index, length, dim, W, b):
    row3 = edge_index[0].reshape(NW, NB, K)
    col3 = edge_index[1].reshape(NW, NB, K)
    col2 = edge_index[1].reshape(NW, EPT)
    ridx = jnp.arange(HR, dtype=jnp.int32).reshape(1, HR)
    zrows = jnp.zeros((HR, D), jnp.float32)
    zeros_nd = jnp.zeros((N, D), jnp.float32)

    deg2 = _deg_kernel(col2, ridx, zrows)
    deg_n1 = (deg2[0] + deg2[1]).reshape(-1)[:N].reshape(N, 1) + 1.0

    h = pl.pallas_call(
        _mm_body,
        grid=(_G,),
        in_specs=[pl.BlockSpec((_RB, D), lambda i: (i, 0)),
                  pl.BlockSpec((D, D), lambda i: (0, 0))],
        out_specs=pl.BlockSpec((_RB, D), lambda i: (i, 0)),
        out_shape=jax.ShapeDtypeStruct((N, D), jnp.float32),
    )(x, W)

    hs = pl.pallas_call(
        _scale_body,
        grid=(_G,),
        in_specs=[pl.BlockSpec((_RB, D), lambda i: (i, 0)),
                  pl.BlockSpec((_RB, 1), lambda i: (i, 0))],
        out_specs=pl.BlockSpec((_RB, D), lambda i: (i, 0)),
        out_shape=jax.ShapeDtypeStruct((N, D), jnp.float32),
    )(h, deg_n1)

    p = _scatter_kernel(row3, col3, hs, zeros_nd)

    out = pl.pallas_call(
        _finish_body,
        grid=(_G,),
        in_specs=[pl.BlockSpec((NC, _RB, D), lambda i: (0, i, 0)),
                  pl.BlockSpec((_RB, D), lambda i: (i, 0)),
                  pl.BlockSpec((_RB, 1), lambda i: (i, 0)),
                  pl.BlockSpec((1, D), lambda i: (0, 0))],
        out_specs=pl.BlockSpec((_RB, D), lambda i: (i, 0)),
        out_shape=jax.ShapeDtypeStruct((N, D), jnp.float32),
    )(p, hs, deg_n1, b.reshape(1, D))
    return out


# speedup vs baseline: 32.2124x; 1.1840x over previous
import functools

import jax
import jax.numpy as jnp
from jax import lax
from jax.experimental import pallas as pl
from jax.experimental.pallas import tpu as pltpu
from jax.experimental.pallas import tpu_sc as plsc

N = 10000
E = 320000
D = 128

NC = 2
NS = 16
NW = NC * NS
EPT = E // NW
K = 80
NB = EPT // K
CH = 1000
NCH = N // CH

_mesh = plsc.VectorSubcoreMesh(core_axis_name="c", subcore_axis_name="s")


NP = 10240
HR = NP // D


@functools.partial(
    pl.kernel,
    out_type=jax.ShapeDtypeStruct((NC, HR, D), jnp.float32),
    mesh=_mesh,
    compiler_params=pltpu.CompilerParams(needs_layout_passes=False),
    scratch_types=[
        pltpu.VMEM((EPT,), jnp.int32),
        pltpu.VMEM((HR, D), jnp.float32),
        pltpu.VMEM((1, HR), jnp.int32),
        pltpu.VMEM_SHARED((HR, D), jnp.float32),
    ],
)
def _deg_kernel(col_hbm, ridx_hbm, zrows_hbm, out_hbm, colb, hist, ridx, dacc):
    c = lax.axis_index("c")
    s = lax.axis_index("s")
    wid = s * NC + c

    @pl.when(s == 0)
    def _init():
        pltpu.sync_copy(zrows_hbm, dacc)

    pltpu.sync_copy(zrows_hbm, hist)
    pltpu.sync_copy(col_hbm.at[wid], colb)
    pltpu.sync_copy(ridx_hbm, ridx)
    ones16 = jnp.ones((16,), jnp.float32)

    def body(i, carry):
        idx = colb[pl.ds(i * 16, 16)]
        r = lax.shift_right_logical(idx, 7)
        q = lax.bitwise_and(idx, 127)
        plsc.addupdate_scatter(hist, [r, q], ones16)
        return carry

    lax.fori_loop(0, EPT // 16, body, 0)
    plsc.subcore_barrier()
    pltpu.sync_copy(hist, dacc.at[ridx.at[0]], add=True)
    plsc.subcore_barrier()

    @pl.when(s == 0)
    def _drain():
        pltpu.sync_copy(dacc, out_hbm.at[c])


CHB = 25
NCK = NB // CHB


@functools.partial(
    pl.kernel,
    out_type=jax.ShapeDtypeStruct((NC, N, D), jnp.float32),
    mesh=_mesh,
    scratch_types=[
        pltpu.VMEM((CHB, K), jnp.int32),
        pltpu.VMEM((CHB, K), jnp.int32),
        pltpu.VMEM((K, D), jnp.float32),
        pltpu.VMEM((K, D), jnp.float32),
        pltpu.VMEM_SHARED((N, D), jnp.float32),
        pltpu.SemaphoreType.DMA,
        pltpu.SemaphoreType.DMA,
    ],
)
def _scatter_kernel(row_hbm, col_hbm, hs_hbm, zeros_hbm, out_hbm,
                    rowv, colv, bufa, bufb, acc, sema, semb):
    c = lax.axis_index("c")
    s = lax.axis_index("s")
    wid = s * NC + c

    @pl.when(s < NCH)
    def _init():
        pltpu.sync_copy(zeros_hbm.at[pl.ds(s * CH, CH)],
                        acc.at[pl.ds(s * CH, CH)])

    plsc.subcore_barrier()

    for ck in range(NCK):
        pltpu.sync_copy(row_hbm.at[wid, ck], rowv)
        pltpu.sync_copy(col_hbm.at[wid, ck], colv)
        pltpu.async_copy(hs_hbm.at[rowv.at[0]], bufa, sema)

        def body(i, carry):
            j = i * 2
            pltpu.make_async_copy(hs_hbm.at[rowv.at[j]], bufa, sema).wait()
            pltpu.async_copy(hs_hbm.at[rowv.at[j + 1]], bufb, semb)
            pltpu.sync_copy(bufa, acc.at[colv.at[j]], add=True)
            pltpu.make_async_copy(hs_hbm.at[rowv.at[j + 1]], bufb, semb).wait()
            pltpu.async_copy(hs_hbm.at[rowv.at[j + 2]], bufa, sema)
            pltpu.sync_copy(bufb, acc.at[colv.at[j + 1]], add=True)
            return carry

        lax.fori_loop(0, (CHB - 1) // 2, body, 0)
        pltpu.make_async_copy(hs_hbm.at[rowv.at[CHB - 1]], bufa, sema).wait()
        pltpu.sync_copy(bufa, acc.at[colv.at[CHB - 1]], add=True)

    plsc.subcore_barrier()

    @pl.when(s < NCH)
    def _drain():
        pltpu.sync_copy(acc.at[pl.ds(s * CH, CH)],
                        out_hbm.at[c, pl.ds(s * CH, CH)])


def _mm_body(x_ref, w_ref, o_ref):
    o_ref[...] = jnp.dot(x_ref[...], w_ref[...],
                         preferred_element_type=jnp.float32)


def _scale_body(h_ref, dg_ref, o_ref):
    o_ref[...] = h_ref[...] * lax.rsqrt(dg_ref[...])


def _finish_body(p_ref, hs_ref, dg_ref, b_ref, o_ref):
    acc = p_ref[0] + p_ref[1] + hs_ref[...]
    t = acc * lax.rsqrt(dg_ref[...]) + b_ref[...]
    t = jnp.maximum(t, 0.0)
    m = jnp.max(t, axis=1, keepdims=True)
    e = jnp.exp(t - m)
    o_ref[...] = e / jnp.sum(e, axis=1, keepdims=True)


_RB = 1000
_G = N // _RB


def kernel(x, edge_index, length, dim, W, b):
    row3 = edge_index[0].reshape(NW, NCK, CHB, K)
    col3 = edge_index[1].reshape(NW, NCK, CHB, K)
    col2 = edge_index[1].reshape(NW, EPT)
    ridx = jnp.arange(HR, dtype=jnp.int32).reshape(1, HR)
    zrows = jnp.zeros((HR, D), jnp.float32)
    zeros_nd = jnp.zeros((N, D), jnp.float32)

    deg2 = _deg_kernel(col2, ridx, zrows)
    deg_n1 = (deg2[0] + deg2[1]).reshape(-1)[:N].reshape(N, 1) + 1.0

    h = pl.pallas_call(
        _mm_body,
        grid=(_G,),
        in_specs=[pl.BlockSpec((_RB, D), lambda i: (i, 0)),
                  pl.BlockSpec((D, D), lambda i: (0, 0))],
        out_specs=pl.BlockSpec((_RB, D), lambda i: (i, 0)),
        out_shape=jax.ShapeDtypeStruct((N, D), jnp.float32),
    )(x, W)

    hs = pl.pallas_call(
        _scale_body,
        grid=(_G,),
        in_specs=[pl.BlockSpec((_RB, D), lambda i: (i, 0)),
                  pl.BlockSpec((_RB, 1), lambda i: (i, 0))],
        out_specs=pl.BlockSpec((_RB, D), lambda i: (i, 0)),
        out_shape=jax.ShapeDtypeStruct((N, D), jnp.float32),
    )(h, deg_n1)

    p = _scatter_kernel(row3, col3, hs, zeros_nd)

    out = pl.pallas_call(
        _finish_body,
        grid=(_G,),
        in_specs=[pl.BlockSpec((NC, _RB, D), lambda i: (0, i, 0)),
                  pl.BlockSpec((_RB, D), lambda i: (i, 0)),
                  pl.BlockSpec((_RB, 1), lambda i: (i, 0)),
                  pl.BlockSpec((1, D), lambda i: (0, 0))],
        out_specs=pl.BlockSpec((_RB, D), lambda i: (i, 0)),
        out_shape=jax.ShapeDtypeStruct((N, D), jnp.float32),
    )(p, hs, deg_n1, b.reshape(1, D))
    return out


# speedup vs baseline: 32.4438x; 1.0072x over previous
import functools

import jax
import jax.numpy as jnp
from jax import lax
from jax.experimental import pallas as pl
from jax.experimental.pallas import tpu as pltpu
from jax.experimental.pallas import tpu_sc as plsc

N = 10000
E = 320000
D = 128

NC = 2
NS = 16
NW = NC * NS
EPT = E // NW
K = 80
NB = EPT // K
CH = 1000
NCH = N // CH

_mesh = plsc.VectorSubcoreMesh(core_axis_name="c", subcore_axis_name="s")


NP = 10240
HR = NP // D


@functools.partial(
    pl.kernel,
    out_type=jax.ShapeDtypeStruct((NC, HR, D), jnp.float32),
    mesh=_mesh,
    compiler_params=pltpu.CompilerParams(needs_layout_passes=False),
    scratch_types=[
        pltpu.VMEM((EPT,), jnp.int32),
        pltpu.VMEM((HR, D), jnp.float32),
        pltpu.VMEM((1, HR), jnp.int32),
        pltpu.VMEM_SHARED((HR, D), jnp.float32),
    ],
)
def _deg_kernel(col_hbm, ridx_hbm, zrows_hbm, out_hbm, colb, hist, ridx, dacc):
    c = lax.axis_index("c")
    s = lax.axis_index("s")
    wid = s * NC + c

    @pl.when(s == 0)
    def _init():
        pltpu.sync_copy(zrows_hbm, dacc)

    pltpu.sync_copy(zrows_hbm, hist)
    pltpu.sync_copy(col_hbm.at[wid], colb)
    pltpu.sync_copy(ridx_hbm, ridx)
    ones16 = jnp.ones((16,), jnp.float32)

    def body(i, carry):
        idx = colb[pl.ds(i * 16, 16)]
        r = lax.shift_right_logical(idx, 7)
        q = lax.bitwise_and(idx, 127)
        plsc.addupdate_scatter(hist, [r, q], ones16)
        return carry

    lax.fori_loop(0, EPT // 16, body, 0)
    plsc.subcore_barrier()
    pltpu.sync_copy(hist, dacc.at[ridx.at[0]], add=True)
    plsc.subcore_barrier()

    @pl.when(s == 0)
    def _drain():
        pltpu.sync_copy(dacc, out_hbm.at[c])


CHB = 25
NCK = NB // CHB


@functools.partial(
    pl.kernel,
    out_type=jax.ShapeDtypeStruct((NC, N, D), jnp.float32),
    mesh=_mesh,
    scratch_types=[
        pltpu.VMEM((CHB, K), jnp.int32),
        pltpu.VMEM((CHB, K), jnp.int32),
        pltpu.VMEM((K, D), jnp.float32),
        pltpu.VMEM((K, D), jnp.float32),
        pltpu.VMEM_SHARED((N, D), jnp.float32),
        pltpu.SemaphoreType.DMA,
        pltpu.SemaphoreType.DMA,
        pltpu.SemaphoreType.DMA,
        pltpu.SemaphoreType.DMA,
    ],
)
def _scatter_kernel(row_hbm, col_hbm, hs_hbm, zeros_hbm, out_hbm,
                    rowv, colv, bufa, bufb, acc, sema, semb, ssa, ssb):
    c = lax.axis_index("c")
    s = lax.axis_index("s")
    wid = s * NC + c

    @pl.when(s < NCH)
    def _init():
        pltpu.sync_copy(zeros_hbm.at[pl.ds(s * CH, CH)],
                        acc.at[pl.ds(s * CH, CH)])

    plsc.subcore_barrier()

    for ck in range(NCK):
        pltpu.sync_copy(row_hbm.at[wid, ck], rowv)
        pltpu.sync_copy(col_hbm.at[wid, ck], colv)
        pltpu.async_copy(hs_hbm.at[rowv.at[0]], bufa, sema)
        pltpu.async_copy(hs_hbm.at[rowv.at[1]], bufb, semb)

        def body(i, carry):
            j = i * 2
            pltpu.make_async_copy(hs_hbm.at[rowv.at[j]], bufa, sema).wait()
            pltpu.async_copy(bufa, acc.at[colv.at[j]], ssa, add=True)
            pltpu.make_async_copy(hs_hbm.at[rowv.at[j + 1]], bufb, semb).wait()
            pltpu.async_copy(bufb, acc.at[colv.at[j + 1]], ssb, add=True)
            pltpu.make_async_copy(bufa, acc.at[colv.at[j]], ssa).wait()
            pltpu.async_copy(hs_hbm.at[rowv.at[j + 2]], bufa, sema)
            pltpu.make_async_copy(bufb, acc.at[colv.at[j + 1]], ssb).wait()
            pltpu.async_copy(hs_hbm.at[rowv.at[j + 3]], bufb, semb)
            return carry

        lax.fori_loop(0, (CHB - 3) // 2, body, 0)
        pltpu.make_async_copy(hs_hbm.at[rowv.at[CHB - 3]], bufa, sema).wait()
        pltpu.sync_copy(bufa, acc.at[colv.at[CHB - 3]], add=True)
        pltpu.make_async_copy(hs_hbm.at[rowv.at[CHB - 2]], bufb, semb).wait()
        pltpu.sync_copy(bufb, acc.at[colv.at[CHB - 2]], add=True)
        pltpu.async_copy(hs_hbm.at[rowv.at[CHB - 1]], bufa, sema).wait()
        pltpu.sync_copy(bufa, acc.at[colv.at[CHB - 1]], add=True)

    plsc.subcore_barrier()

    @pl.when(s < NCH)
    def _drain():
        pltpu.sync_copy(acc.at[pl.ds(s * CH, CH)],
                        out_hbm.at[c, pl.ds(s * CH, CH)])


def _mm_body(x_ref, w_ref, dg_ref, o_ref):
    h = jnp.dot(x_ref[...], w_ref[...], preferred_element_type=jnp.float32)
    o_ref[...] = h * lax.rsqrt(dg_ref[...])


def _finish_body(p_ref, hs_ref, dg_ref, b_ref, o_ref):
    acc = p_ref[0] + p_ref[1] + hs_ref[...]
    t = acc * lax.rsqrt(dg_ref[...]) + b_ref[...]
    t = jnp.maximum(t, 0.0)
    m = jnp.max(t, axis=1, keepdims=True)
    e = jnp.exp(t - m)
    o_ref[...] = e / jnp.sum(e, axis=1, keepdims=True)


_RB = 1000
_G = N // _RB


def kernel(x, edge_index, length, dim, W, b):
    row3 = edge_index[0].reshape(NW, NCK, CHB, K)
    col3 = edge_index[1].reshape(NW, NCK, CHB, K)
    col2 = edge_index[1].reshape(NW, EPT)
    ridx = jnp.arange(HR, dtype=jnp.int32).reshape(1, HR)
    zrows = jnp.zeros((HR, D), jnp.float32)
    zeros_nd = jnp.zeros((N, D), jnp.float32)

    deg2 = _deg_kernel(col2, ridx, zrows)
    deg_n1 = (deg2[0] + deg2[1]).reshape(-1)[:N].reshape(N, 1) + 1.0

    hs = pl.pallas_call(
        _mm_body,
        grid=(_G,),
        in_specs=[pl.BlockSpec((_RB, D), lambda i: (i, 0)),
                  pl.BlockSpec((D, D), lambda i: (0, 0)),
                  pl.BlockSpec((_RB, 1), lambda i: (i, 0))],
        out_specs=pl.BlockSpec((_RB, D), lambda i: (i, 0)),
        out_shape=jax.ShapeDtypeStruct((N, D), jnp.float32),
    )(x, W, deg_n1)

    p = _scatter_kernel(row3, col3, hs, zeros_nd)

    out = pl.pallas_call(
        _finish_body,
        grid=(_G,),
        in_specs=[pl.BlockSpec((NC, _RB, D), lambda i: (0, i, 0)),
                  pl.BlockSpec((_RB, D), lambda i: (i, 0)),
                  pl.BlockSpec((_RB, 1), lambda i: (i, 0)),
                  pl.BlockSpec((1, D), lambda i: (0, 0))],
        out_specs=pl.BlockSpec((_RB, D), lambda i: (i, 0)),
        out_shape=jax.ShapeDtypeStruct((N, D), jnp.float32),
    )(p, hs, deg_n1, b.reshape(1, D))
    return out
